# Initial kernel scaffold; baseline (speedup 1.0000x reference)
import jax
import jax.numpy as jnp
from jax.experimental import pallas as pl


def kernel(ngrams, table):
    raise NotImplementedError("write your pallas kernel here")


# speedup vs baseline: 38.3286x; 38.3286x over previous
import functools

import jax
import jax.numpy as jnp
from jax import lax
from jax.experimental import pallas as pl
from jax.experimental.pallas import tpu as pltpu
from jax.experimental.pallas import tpu_sc as plsc

_B = 4096
_L = 50
_D = 32
_NC = 2
_NS = 16
_NW = _NC * _NS
_BPW = _B // _NW
_CB = 16
_NCHUNK = _BPW // _CB
_ROWS = _CB * _L


def _embed_body(flat_hbm, table_hbm, out_hbm, idx_v, rows_v, acc_v, sem):
    wid = lax.axis_index("s") * _NC + lax.axis_index("c")
    bag_base = wid * _BPW

    def chunk_body(c, carry):
        bag0 = bag_base + c * _CB
        pltpu.sync_copy(flat_hbm.at[pl.ds(bag0 * _L, _ROWS)], idx_v)
        pltpu.async_copy(table_hbm.at[idx_v], rows_v, sem).wait()

        def bag_body(b, carry2):
            def l_body(l, accs):
                a0, a1 = accs
                r = b * _L + l
                a0 = a0 + rows_v[r, pl.ds(0, 16)]
                a1 = a1 + rows_v[r, pl.ds(16, 16)]
                return (a0, a1)

            z = jnp.zeros((16,), jnp.float32)
            a0, a1 = lax.fori_loop(0, _L, l_body, (z, z))
            acc_v[b, pl.ds(0, 16)] = a0
            acc_v[b, pl.ds(16, 16)] = a1
            return carry2

        lax.fori_loop(0, _CB, bag_body, 0)
        pltpu.sync_copy(acc_v, out_hbm.at[pl.ds(bag0, _CB)])
        return carry

    lax.fori_loop(0, _NCHUNK, chunk_body, 0)


def kernel(ngrams, table):
    flat = ngrams.reshape(-1)
    mesh = plsc.VectorSubcoreMesh(core_axis_name="c", subcore_axis_name="s")
    k = functools.partial(
        pl.kernel,
        mesh=mesh,
        out_type=jax.ShapeDtypeStruct((_B, _D), jnp.float32),
        scratch_types=[
            pltpu.VMEM((_ROWS,), jnp.int32),
            pltpu.VMEM((_ROWS, _D), jnp.float32),
            pltpu.VMEM((_CB, _D), jnp.float32),
            pltpu.SemaphoreType.DMA,
        ],
        compiler_params=pltpu.CompilerParams(use_tc_tiling_on_sc=False),
    )(_embed_body)
    return k(flat, table)


# speedup vs baseline: 40.0329x; 1.0445x over previous
import functools

import jax
import jax.numpy as jnp
from jax import lax
from jax.experimental import pallas as pl
from jax.experimental.pallas import tpu as pltpu
from jax.experimental.pallas import tpu_sc as plsc

_B = 4096
_L = 50
_D = 32
_NC = 2
_NS = 16
_NW = _NC * _NS
_BPW = _B // _NW
_CB = 16
_NCHUNK = _BPW // _CB
_ROWS = _CB * _L


def _embed_body(flat_hbm, table_hbm, out_hbm, idx_v, rows_v, acc_v, gsem, osem):
    wid = lax.axis_index("s") * _NC + lax.axis_index("c")
    bag_base = wid * _BPW

    def issue_gather(c, buf):
        bag0 = bag_base + c * _CB
        pltpu.sync_copy(flat_hbm.at[pl.ds(bag0 * _L, _ROWS)], idx_v.at[buf])
        return pltpu.async_copy(table_hbm.at[idx_v.at[buf]], rows_v.at[buf], gsem)

    gathers = {0: issue_gather(0, 0)}
    writebacks = {}
    for c in range(_NCHUNK):
        buf = c % 2
        if c + 1 < _NCHUNK:
            gathers[c + 1] = issue_gather(c + 1, 1 - buf)
        gathers[c].wait()
        if c - 2 in writebacks:
            writebacks[c - 2].wait()

        def bag_body(b, carry, _buf=buf):
            base = b * _L
            r = rows_v
            a0 = r[_buf, base, pl.ds(0, 16)]
            a1 = r[_buf, base, pl.ds(16, 16)]
            b0 = r[_buf, base + 1, pl.ds(0, 16)]
            b1 = r[_buf, base + 1, pl.ds(16, 16)]
            for l in range(2, _L, 2):
                a0 = a0 + r[_buf, base + l, pl.ds(0, 16)]
                a1 = a1 + r[_buf, base + l, pl.ds(16, 16)]
                b0 = b0 + r[_buf, base + l + 1, pl.ds(0, 16)]
                b1 = b1 + r[_buf, base + l + 1, pl.ds(16, 16)]
            acc_v[_buf, b, pl.ds(0, 16)] = a0 + b0
            acc_v[_buf, b, pl.ds(16, 16)] = a1 + b1
            return carry

        lax.fori_loop(0, _CB, bag_body, 0)
        writebacks[c] = pltpu.async_copy(
            acc_v.at[buf], out_hbm.at[pl.ds(bag_base + c * _CB, _CB)], osem)
    writebacks[_NCHUNK - 2].wait()
    writebacks[_NCHUNK - 1].wait()


def kernel(ngrams, table):
    flat = ngrams.reshape(-1)
    mesh = plsc.VectorSubcoreMesh(core_axis_name="c", subcore_axis_name="s")
    k = functools.partial(
        pl.kernel,
        mesh=mesh,
        out_type=jax.ShapeDtypeStruct((_B, _D), jnp.float32),
        scratch_types=[
            pltpu.VMEM((2, _ROWS), jnp.int32),
            pltpu.VMEM((2, _ROWS, _D), jnp.float32),
            pltpu.VMEM((2, _CB, _D), jnp.float32),
            pltpu.SemaphoreType.DMA,
            pltpu.SemaphoreType.DMA,
        ],
        compiler_params=pltpu.CompilerParams(use_tc_tiling_on_sc=False),
    )(_embed_body)
    return k(flat, table)
